# baseline (speedup 1.0000x reference)
import jax
import jax.numpy as jnp
from jax.experimental import pallas as pl
from jax.experimental.pallas import tpu as pltpu

_B = 16384
_CHUNK = 1024
_NCHUNK = _B // _CHUNK
_NBUF = 4
_VIEW = (128, 256, 512)


def _pack_layout():
    ofs = {}
    row = 0
    for v in range(3):
        vs = _VIEW[v]
        nb = vs // 128
        for name, nrows in (("Wt", vs), ("Wa", 128), ("Wb", 128), ("Wc", 64),
                            ("Wd1", 128), ("Wd2", vs)):
            ofs[(v, name)] = row
            row += nrows
        nbias = 5 + nb
        nbias_pad = ((nbias + 7) // 8) * 8
        ofs[(v, "bias")] = row
        row += nbias_pad
    return ofs, row


_OFS, _NROWS = _pack_layout()


def _body(*refs):
    xs = refs[0:3]
    w = refs[3]
    outs = refs[4:13]
    (xb0, xb1, xb2, rxb0, rxb1, rxb2, zcb0, zcb1, zcb2, zsb0, zsb1, zsb2,
     sin, srx, szc, szs) = refs[13:]
    zcbufs = (zcb0, zcb1, zcb2)
    zsbufs = (zsb0, zsb1, zsb2)
    xbufs = (xb0, xb1, xb2)
    rxbufs = (rxb0, rxb1, rxb2)

    def in_copy(i):
        slot = i % _NBUF
        return [pltpu.make_async_copy(
            xs[v].at[pl.ds(i * _CHUNK, _CHUNK), :], xbufs[v].at[slot], sin.at[slot, v])
            for v in range(3)]

    def rx_copy(i):
        slot = i % _NBUF
        cps = [pltpu.make_async_copy(
            rxbufs[v].at[slot], outs[6 + v].at[pl.ds(i * _CHUNK, _CHUNK), :], srx.at[slot, v])
            for v in range(3)]
        cps += [pltpu.make_async_copy(
            zcbufs[v].at[slot], outs[v].at[pl.ds(i * _CHUNK, _CHUNK), :], szc.at[slot, v])
            for v in range(3)]
        cps += [pltpu.make_async_copy(
            zsbufs[v].at[slot], outs[3 + v].at[pl.ds(i * _CHUNK, _CHUNK), :], szs.at[slot, v])
            for v in range(3)]
        return cps

    def compute(i):
        slot = i % _NBUF
        for v in range(3):
            vs = _VIEW[v]
            nb = vs // 128
            o = {n: _OFS[(v, n)] for n in ("Wt", "Wa", "Wb", "Wc", "Wd1", "Wd2", "bias")}
            ob = o["bias"]
            x = xbufs[v][slot]
            z1 = jnp.dot(x, w[o["Wt"]:o["Wt"] + vs, :],
                         preferred_element_type=jnp.float32) + w[ob, :][None, :]
            h = jnp.maximum(jnp.dot(z1, w[o["Wa"]:o["Wa"] + 128, :],
                                    preferred_element_type=jnp.float32)
                            + w[ob + 1, :][None, :], 0.0)
            z = jnp.dot(h, w[o["Wb"]:o["Wb"] + 128, :64],
                        preferred_element_type=jnp.float32) + w[ob + 2, :64][None, :]
            d = jnp.dot(z, w[o["Wc"]:o["Wc"] + 64, :],
                        preferred_element_type=jnp.float32) + w[ob + 3, :][None, :]
            g = jnp.maximum(jnp.dot(d, w[o["Wd1"]:o["Wd1"] + 128, :],
                                    preferred_element_type=jnp.float32)
                            + w[ob + 4, :][None, :], 0.0)
            zcbufs[v][slot] = z[:, :32]
            zsbufs[v][slot] = z[:, 32:]
            for j in range(nb):
                rxbufs[v][slot, :, j * 128:(j + 1) * 128] = (
                    jnp.dot(g, w[o["Wd2"] + j * 128:o["Wd2"] + (j + 1) * 128, :],
                            preferred_element_type=jnp.float32)
                    + w[ob + 5 + j, :][None, :])

    for i in range(min(_NBUF, _NCHUNK)):
        for c in in_copy(i):
            c.start()
    for i in range(_NCHUNK):
        for c in in_copy(i):
            c.wait()
        if i >= _NBUF:
            for c in rx_copy(i - _NBUF):
                c.wait()
        compute(i)
        for c in rx_copy(i):
            c.start()
        if i + _NBUF < _NCHUNK:
            for c in in_copy(i + _NBUF):
                c.start()
    for i in range(max(_NCHUNK - _NBUF, 0), _NCHUNK):
        for c in rx_copy(i):
            c.wait()


def kernel(x0, x1, x2, trunk_params, content_params, style_params,
           dec_content_params, dec_style_params, dec_trunk_params, status=0):
    xs = (x0, x1, x2)
    pieces = []
    for v in range(3):
        vs = _VIEW[v]
        nb = vs // 128
        (Wt, bt), = trunk_params[v]
        (Wc1, bc1), (Wc2, bc2) = content_params[v]
        (Ws1, bs1), (Ws2, bs2) = style_params[v]
        (Wdc, bdc), = dec_content_params[v]
        (Wds, bds), = dec_style_params[v]
        (Wd1, bd1), (Wd2, bd2) = dec_trunk_params[v]
        z64 = jnp.zeros((64, 32), jnp.float32)
        z32 = jnp.zeros((32, 64), jnp.float32)
        Wa = jnp.concatenate([Wc1, Ws1], axis=1)
        Wb = jnp.block([[Wc2, z64], [z64, Ws2]])
        Wcc = jnp.block([[Wdc, z32], [z32, Wds]])
        Wd2s = jnp.transpose(Wd2.reshape(128, nb, 128), (1, 0, 2)).reshape(nb * 128, 128)
        bias_rows = jnp.stack([
            bt,
            jnp.concatenate([bc1, bs1]),
            jnp.pad(jnp.concatenate([bc2, bs2]), (0, 64)),
            jnp.concatenate([bdc, bds]),
            bd1,
        ] + [bd2[j * 128:(j + 1) * 128] for j in range(nb)])
        nbias = 5 + nb
        nbias_pad = ((nbias + 7) // 8) * 8
        bias_rows = jnp.pad(bias_rows, ((0, nbias_pad - nbias), (0, 0)))
        Wb_pad = jnp.pad(Wb, ((0, 0), (0, 64)))
        pieces += [Wt, Wa, Wb_pad, Wcc, Wd1, Wd2s, bias_rows]
    packed = jnp.concatenate(pieces, axis=0)

    any_spec = pl.BlockSpec(memory_space=pl.ANY)
    out_shape = (
        [jax.ShapeDtypeStruct((_B, 32), jnp.float32) for _ in range(6)]
        + [jax.ShapeDtypeStruct((_B, _VIEW[v]), jnp.float32) for v in range(3)]
    )
    scratch = (
        [pltpu.VMEM((_NBUF, _CHUNK, _VIEW[v]), jnp.float32) for v in range(3)]
        + [pltpu.VMEM((_NBUF, _CHUNK, _VIEW[v]), jnp.float32) for v in range(3)]
        + [pltpu.VMEM((_NBUF, _CHUNK, 32), jnp.float32) for _ in range(6)]
        + [pltpu.SemaphoreType.DMA((_NBUF, 3))] * 4
    )
    outs = pl.pallas_call(
        _body,
        in_specs=[any_spec] * 3 + [pl.BlockSpec(memory_space=pltpu.MemorySpace.VMEM)],
        out_specs=[any_spec] * 9,
        out_shape=out_shape,
        scratch_shapes=scratch,
    )(*xs, packed)
    return tuple(outs)
